# baseline (speedup 1.0000x reference)
import functools

import numpy as np
import jax
import jax.numpy as jnp
from jax.experimental import pallas as pl
from jax.experimental.pallas import tpu as pltpu

_C1 = np.float32(0.0001)
_C2 = np.float32(0.0005)
_LANES = 128


def _f32(x):
    return np.asarray(x, np.float32)


def _np_linspace_f32(a, b, n):
    return np.linspace(a, b, n).astype(np.float32)


def _warp_blend_taps(h):
    yb = _np_linspace_f32(0.0, 1.0, h)
    gy = (np.float32(2.0) * yb - np.float32(1.0)).astype(np.float32)
    iy = ((gy + np.float32(1.0)) * np.float32(h) - np.float32(1.0)) * np.float32(0.5)
    y0 = np.floor(iy).astype(np.int64)
    wy = (iy - y0.astype(np.float32)).astype(np.float32)
    taps = np.zeros((h, 3), np.float32)
    for r in range(h):
        for yy, ww in ((y0[r], np.float32(1.0) - wy[r]), (y0[r] + 1, wy[r])):
            if 0 <= yy < h:
                t = int(yy - r + 1)
                assert 0 <= t <= 2
                taps[r, t] += ww
    return taps


def _resize_vert_taps(n_in, n_out, f):
    ys = _np_linspace_f32(0.0, n_in - 1.0, n_out)
    y0 = np.floor(ys).astype(np.int64)
    y1 = np.minimum(y0 + 1, n_in - 1)
    wy = (ys - y0.astype(np.float32)).astype(np.float32)
    taps = np.zeros((n_out, f + 1), np.float32)
    for i in range(n_out):
        for yy, ww in ((y0[i], np.float32(1.0) - wy[i]), (y1[i], wy[i])):
            t = int(yy - f * i)
            assert 0 <= t <= f, (i, yy, f)
            taps[i, t] += ww
    return taps


def _resize_horiz_idx(n_in, n_out):
    xs = _np_linspace_f32(0.0, n_in - 1.0, n_out)
    x0 = np.floor(xs).astype(np.int64)
    x1 = np.minimum(x0 + 1, n_in - 1)
    wx = (xs - x0.astype(np.float32)).astype(np.float32)
    return x0.astype(np.int32), x1.astype(np.int32), wx


def _static_ranges(idx, nb_out, bw, nb_src):
    out = []
    for o in range(nb_out):
        seg = idx[o * bw:(o + 1) * bw]
        out.append((max(0, int(seg.min()) // bw), min(nb_src - 1, int(seg.max()) // bw)))
    return out


def _dyn_ranges(kind, nb):
    out = []
    for o in range(nb):
        if kind == "neg":
            lo, hi = 0, o
        else:
            lo, hi = o, nb - 1
        out.append((lo, hi))
    return out


def _shl(x, k):
    r, c = x.shape
    return jnp.concatenate([x[:, k:], jnp.zeros((r, k), jnp.float32)], axis=1)


def _shu(x, k=1):
    r, c = x.shape
    return jnp.concatenate([x[k:, :], jnp.zeros((k, c), jnp.float32)], axis=0)


def _shd(x, k=1):
    r, c = x.shape
    return jnp.concatenate([jnp.zeros((k, c), jnp.float32), x[:-k, :]], axis=0)


def _rl(x, k):
    return pltpu.roll(x, x.shape[1] - k, 1)


def _ru(x, k):
    return pltpu.roll(x, x.shape[0] - k, 0)


def _rd(x, k):
    return pltpu.roll(x, k, 0)


def _lane_gather(x, idx, ranges, bw):
    rows, src_w = x.shape
    _, out_w = idx.shape
    nb_out = max(1, out_w // bw)
    cols = []
    for o in range(nb_out):
        if nb_out > 1:
            io = idx[:, o * bw:(o + 1) * bw]
        else:
            io = idx
        acc = None
        lo, hi = ranges[o]
        for s in range(lo, hi + 1):
            src = x[:, s * bw:(s + 1) * bw]
            rel = io - s * bw
            g = jnp.take_along_axis(src, jnp.clip(rel, 0, bw - 1), axis=1)
            t = jnp.where((rel >= 0) & (rel < bw), g, jnp.float32(0.0))
            acc = t if acc is None else acc + t
        cols.append(acc)
    return jnp.concatenate(cols, axis=1) if nb_out > 1 else cols[0]


def _lane_gather_lerp(x, d, idx, frac, ranges, bw, boundary=True, neg=False):
    rows, src_w = x.shape
    _, out_w = idx.shape
    nb_out = max(1, out_w // bw)
    cols = []
    for o in range(nb_out):
        if nb_out > 1:
            io = idx[:, o * bw:(o + 1) * bw]
            fo = frac[:, o * bw:(o + 1) * bw]
        else:
            io, fo = idx, frac
        acc = None
        lo, hi = ranges[o]
        for s in range(lo, hi + 1):
            rel = io - s * bw
            crel = jnp.clip(rel, 0, bw - 1)
            gx = jnp.take_along_axis(x[:, s * bw:(s + 1) * bw], crel, axis=1)
            gd = jnp.take_along_axis(d[:, s * bw:(s + 1) * bw], crel, axis=1)
            if neg and s == o:
                inb = rel >= 0
            else:
                inb = (rel >= 0) & (rel < bw)
            t = jnp.where(inb, gx + fo * gd, jnp.float32(0.0))
            acc = t if acc is None else acc + t
        if boundary:
            c0 = lo * bw - 1
            if c0 < 0:
                bval = fo * x[:, 0:1]
            else:
                bval = x[:, c0:c0 + 1] + fo * d[:, c0:c0 + 1]
            acc = acc + jnp.where(io == c0, bval, jnp.float32(0.0))
        cols.append(acc)
    return jnp.concatenate(cols, axis=1) if nb_out > 1 else cols[0]


def _pool9(x):
    hs = x + _rl(x, 1) + _rl(x, 2)
    vs = hs + _ru(hs, 1) + _ru(hs, 2)
    return vs * jnp.float32(1.0 / 9.0)


def _scale_kernel(il_ref, ir_ref, disp_ref, rc_ref, xw_ref, out_ref,
                  sl_ref, sbl_ref, sr_ref, sbr_ref,
                  *, s, h, w, f, big_w, R, K, bw, hranges0):
    nb = max(1, w // bw)

    w0 = rc_ref[:, 0:1]
    w1 = rc_ref[:, 1:2]
    w2 = rc_ref[:, 2:3]

    def blend(x):
        return w0 * _rd(x, 1) + w1 * x + w2 * _ru(x, 1)

    if s > 0:
        x0h = xw_ref[1:2, :].astype(jnp.int32)
        wxh = xw_ref[3:4, :]

    def resized(ref, c):
        if s == 0:
            return ref[0, c]
        v = None
        for k in range(f):
            pk = ref[0, c, :, k, :]
            term = rc_ref[:, 3 + k:4 + k] * pk
            v = term if v is None else v + term
            if k == 0:
                p0 = pk
        v = v + rc_ref[:, 3 + f:4 + f] * _ru(p0, 1)
        dv = _shl(v, 1) - v
        i0 = jnp.broadcast_to(x0h, (h, w))
        f0 = jnp.broadcast_to(wxh, (h, w))
        return _lane_gather_lerp(v, dv, i0, f0, hranges0, bw, boundary=False)

    for c in range(3):
        o_l = resized(il_ref, c)
        sl_ref[c * h:(c + 1) * h, :] = o_l
        sbl_ref[c * h:(c + 1) * h, :] = blend(o_l)
        o_r = resized(ir_ref, c)
        sr_ref[c * h:(c + 1) * h, :] = o_r
        sbr_ref[c * h:(c + 1) * h, :] = blend(o_r)
    dl = -disp_ref[0, 0]
    dr = disp_ref[0, 1]
    sl_ref[3 * h:4 * h, :] = dr
    sbl_ref[3 * h:4 * h, :] = blend(dr)
    sr_ref[3 * h:4 * h, :] = dl
    sbr_ref[3 * h:4 * h, :] = blend(dl)
    zpad = jnp.zeros((8, w), jnp.float32)
    sl_ref[4 * h:4 * h + 8, :] = zpad
    sbl_ref[4 * h:4 * h + 8, :] = zpad
    sr_ref[4 * h:4 * h + 8, :] = zpad
    sbr_ref[4 * h:4 * h + 8, :] = zpad

    xb = xw_ref[0:1, :]
    iota_k = jax.lax.broadcasted_iota(jnp.int32, (K, 1), 0)
    lane_i = jax.lax.broadcasted_iota(jnp.int32, (1, w), 1)
    lm2 = lane_i < (w - 2)
    lm1f = (lane_i < (w - 1)).astype(jnp.float32)
    lane8 = jax.lax.broadcasted_iota(jnp.int32, (1, _LANES), 1)

    inv_wm1 = jnp.float32(1.0 / (w - 1))
    inv_w = jnp.float32(1.0 / w)

    def chunk(cidx, acc):
        r0 = pl.multiple_of(cidx * R, 8)

        def load4(ref):
            return [ref[pl.ds(c * h + r0, K), :] for c in range(4)]

        gl = load4(sl_ref)
        gr = load4(sr_ref)
        bl = jnp.concatenate(load4(sbl_ref), axis=0)
        br = jnp.concatenate(load4(sbr_ref), axis=0)
        drw = gl[3]
        dlw = gr[3]

        def warp_idx(d):
            gx = jnp.float32(2.0) * (xb + d) - jnp.float32(1.0)
            ix = ((gx + jnp.float32(1.0)) * jnp.float32(w) - jnp.float32(1.0)) * jnp.float32(0.5)
            x0f = jnp.floor(ix)
            return x0f.astype(jnp.int32), ix - x0f

        x0l, fxl = warp_idx(dlw)
        x0r, fxr = warp_idx(drw)

        def warp(src4, x0, fx, kind):
            i4 = jnp.concatenate([x0] * 4, axis=0)
            f4 = jnp.concatenate([fx] * 4, axis=0)
            d4 = _shl(src4, 1) - src4
            return _lane_gather_lerp(src4, d4, i4, f4, _dyn_ranges(kind, nb), bw,
                                     neg=(kind == "neg"))

        est_l = warp(bl, x0l, fxl, "neg")
        est_r = warp(br, x0r, fxr, "pos")

        m_ssim = (iota_k < R) & ((iota_k + r0) < (h - 2))
        m3 = jnp.concatenate([m_ssim] * 3, axis=0) & lm2

        def ssim_sum(gt3, e3):
            mu_g = _pool9(gt3)
            mu_e = _pool9(e3)
            sig_g = _pool9(gt3 * gt3) - mu_g * mu_g
            sig_e = _pool9(e3 * e3) - mu_e * mu_e
            sig_ge = _pool9(gt3 * e3) - mu_e * mu_g
            num = (jnp.float32(2.0) * mu_e * mu_g + _C1) * (jnp.float32(2.0) * sig_ge + _C2)
            den = (mu_g * mu_g + mu_e * mu_e + _C1) * (sig_g + sig_e + _C2)
            ssim = num / den
            return jnp.sum(jnp.where(m3, ssim, jnp.float32(0.0)))

        s_ssim_l = ssim_sum(jnp.concatenate(gl[:3], axis=0), est_l[0:3 * K])
        s_ssim_r = ssim_sum(jnp.concatenate(gr[:3], axis=0), est_r[0:3 * K])

        m_row = (iota_k < R).astype(jnp.float32)
        m_dy = ((iota_k < R) & ((iota_k + r0) < (h - 1))).astype(jnp.float32)

        def smooth_sums(dw, img3):
            sdx = jnp.sum(jnp.abs(dw - _rl(dw, 1)) * lm1f, axis=1, keepdims=True)
            sdy = jnp.sum(jnp.abs(dw - _ru(dw, 1)), axis=1, keepdims=True)
            swx = None
            swy = None
            for c in range(3):
                rx = jnp.sum(jnp.abs(img3[c] - _rl(img3[c], 1)) * lm1f,
                             axis=1, keepdims=True)
                ry = jnp.sum(jnp.abs(img3[c] - _ru(img3[c], 1)), axis=1, keepdims=True)
                ex = jnp.exp(-rx * inv_wm1)
                ey = jnp.exp(-ry * inv_w)
                swx = ex if swx is None else swx + ex
                swy = ey if swy is None else swy + ey
            sx = jnp.sum(sdx * swx * m_row)
            sy = jnp.sum(sdy * swy * m_dy)
            return sx, sy

        s_xl, s_yl = smooth_sums(dlw, gl)
        s_xr, s_yr = smooth_sums(drw, gr)

        mrow2 = m_row
        s_lrc_r = jnp.sum(jnp.abs(drw - est_l[3 * K:4 * K]) * mrow2)
        s_lrc_l = jnp.sum(jnp.abs(dlw - est_r[3 * K:4 * K]) * mrow2)

        vals = [s_ssim_l, s_ssim_r, s_xl, s_yl, s_xr, s_yr, s_lrc_r, s_lrc_l]
        upd = acc
        for k, v in enumerate(vals):
            upd = upd + jnp.where(lane8 == k, v, jnp.float32(0.0))
        return upd

    acc = jax.lax.fori_loop(0, h // R, chunk, jnp.zeros((1, _LANES), jnp.float32))
    out_ref[0] = acc


def _scale_call(s, il, ir, disp):
    B, _, H, W = il.shape
    h, w, f = H >> s, W >> s, 1 << s
    R = min(64, h)
    K = R + 8
    bw = min(w, _LANES)
    nb = max(1, w // bw)

    rc = np.zeros((h, 16), np.float32)
    rc[:, 0:3] = _warp_blend_taps(h)
    if s > 0:
        rc[:, 3:3 + f + 1] = _resize_vert_taps(H, h, f)

    xw = np.zeros((8, w), np.float32)
    xw[0] = _np_linspace_f32(0.0, 1.0, w)
    hranges0 = ((0, 0),)
    if s > 0:
        x0h, x1h, wxh = _resize_horiz_idx(W, w)
        xw[1] = x0h.astype(np.float32)
        xw[2] = x1h.astype(np.float32)
        xw[3] = wxh
        nb_src = W // bw
        hranges0 = tuple(_static_ranges(x0h, nb, bw, nb_src))
        il_in = il.reshape(B, 3, h, f, W)
        ir_in = ir.reshape(B, 3, h, f, W)
        img_block = (1, 3, h, f, W)
        img_map = lambda b: (b, 0, 0, 0, 0)
    else:
        il_in, ir_in = il, ir
        img_block = (1, 3, H, W)
        img_map = lambda b: (b, 0, 0, 0)

    body = functools.partial(
        _scale_kernel, s=s, h=h, w=w, f=f, big_w=W, R=R, K=K, bw=bw,
        hranges0=hranges0)

    out = pl.pallas_call(
        body,
        grid=(B,),
        in_specs=[
            pl.BlockSpec(img_block, img_map),
            pl.BlockSpec(img_block, img_map),
            pl.BlockSpec((1, 2, h, w), lambda b: (b, 0, 0, 0)),
            pl.BlockSpec((h, 16), lambda b: (0, 0)),
            pl.BlockSpec((8, w), lambda b: (0, 0)),
        ],
        out_specs=pl.BlockSpec((1, 1, _LANES), lambda b: (b, 0, 0)),
        out_shape=jax.ShapeDtypeStruct((B, 1, _LANES), jnp.float32),
        scratch_shapes=[pltpu.VMEM((4 * h + 8, w), jnp.float32)
                        for _ in range(4)],
        compiler_params=pltpu.CompilerParams(
            dimension_semantics=("parallel",)),
    )(il_in, ir_in, disp, jnp.asarray(rc), jnp.asarray(xw))
    return jnp.sum(out[:, 0, :8], axis=0)


def kernel(images_left, images_right, disp0, disp1, disp2, disp3):
    B, _, H, W = images_left.shape
    disps = [disp0, disp1, disp2, disp3]
    apps, smooths, lrcs = [], [], []
    for s in range(4):
        h, w = H >> s, W >> s
        S = _scale_call(s, images_left, images_right, disps[s])
        n_ssim = np.float32(B * 3 * (h - 2) * (w - 2))
        n_x = np.float32(B * 3 * h * (w - 1))
        n_y = np.float32(B * 3 * (h - 1) * w)
        n_d = np.float32(B * h * w)
        apps.append((S[0] + S[1]) / n_ssim)
        smooths.append(((S[2] + S[4]) / n_x + (S[3] + S[5]) / n_y)
                       / np.float32((s + 1) ** 2))
        lrcs.append((S[6] + S[7]) / n_d)
    return jnp.stack([
        jnp.mean(jnp.stack(apps)),
        jnp.mean(jnp.stack(smooths)),
        jnp.mean(jnp.stack(lrcs)),
    ])


# speedup vs baseline: 1.0704x; 1.0704x over previous
import functools

import numpy as np
import jax
import jax.numpy as jnp
from jax.experimental import pallas as pl
from jax.experimental.pallas import tpu as pltpu

_C1 = np.float32(0.0001)
_C2 = np.float32(0.0005)
_LANES = 128


def _f32(x):
    return np.asarray(x, np.float32)


def _np_linspace_f32(a, b, n):
    return np.linspace(a, b, n).astype(np.float32)


def _warp_blend_taps(h):
    yb = _np_linspace_f32(0.0, 1.0, h)
    gy = (np.float32(2.0) * yb - np.float32(1.0)).astype(np.float32)
    iy = ((gy + np.float32(1.0)) * np.float32(h) - np.float32(1.0)) * np.float32(0.5)
    y0 = np.floor(iy).astype(np.int64)
    wy = (iy - y0.astype(np.float32)).astype(np.float32)
    taps = np.zeros((h, 3), np.float32)
    for r in range(h):
        for yy, ww in ((y0[r], np.float32(1.0) - wy[r]), (y0[r] + 1, wy[r])):
            if 0 <= yy < h:
                t = int(yy - r + 1)
                assert 0 <= t <= 2
                taps[r, t] += ww
    return taps


def _resize_vert_taps(n_in, n_out, f):
    ys = _np_linspace_f32(0.0, n_in - 1.0, n_out)
    y0 = np.floor(ys).astype(np.int64)
    y1 = np.minimum(y0 + 1, n_in - 1)
    wy = (ys - y0.astype(np.float32)).astype(np.float32)
    taps = np.zeros((n_out, f + 1), np.float32)
    for i in range(n_out):
        for yy, ww in ((y0[i], np.float32(1.0) - wy[i]), (y1[i], wy[i])):
            t = int(yy - f * i)
            assert 0 <= t <= f, (i, yy, f)
            taps[i, t] += ww
    return taps


def _resize_horiz_idx(n_in, n_out):
    xs = _np_linspace_f32(0.0, n_in - 1.0, n_out)
    x0 = np.floor(xs).astype(np.int64)
    x1 = np.minimum(x0 + 1, n_in - 1)
    wx = (xs - x0.astype(np.float32)).astype(np.float32)
    return x0.astype(np.int32), x1.astype(np.int32), wx


def _static_ranges(idx, nb_out, bw, nb_src):
    out = []
    for o in range(nb_out):
        seg = idx[o * bw:(o + 1) * bw]
        out.append((max(0, int(seg.min()) // bw), min(nb_src - 1, int(seg.max()) // bw)))
    return out


def _dyn_ranges(kind, nb):
    out = []
    for o in range(nb):
        if kind == "neg":
            lo, hi = 0, o
        else:
            lo, hi = o, nb - 1
        out.append((lo, hi))
    return out


def _shl(x, k):
    r, c = x.shape
    return jnp.concatenate([x[:, k:], jnp.zeros((r, k), jnp.float32)], axis=1)


def _shu(x, k=1):
    r, c = x.shape
    return jnp.concatenate([x[k:, :], jnp.zeros((k, c), jnp.float32)], axis=0)


def _shd(x, k=1):
    r, c = x.shape
    return jnp.concatenate([jnp.zeros((k, c), jnp.float32), x[:-k, :]], axis=0)


def _rl(x, k):
    return pltpu.roll(x, x.shape[1] - k, 1)


def _ru(x, k):
    return pltpu.roll(x, x.shape[0] - k, 0)


def _rd(x, k):
    return pltpu.roll(x, k, 0)


def _lane_gather(x, idx, ranges, bw):
    rows, src_w = x.shape
    _, out_w = idx.shape
    nb_out = max(1, out_w // bw)
    cols = []
    for o in range(nb_out):
        if nb_out > 1:
            io = idx[:, o * bw:(o + 1) * bw]
        else:
            io = idx
        acc = None
        lo, hi = ranges[o]
        for s in range(lo, hi + 1):
            src = x[:, s * bw:(s + 1) * bw]
            rel = io - s * bw
            g = jnp.take_along_axis(src, jnp.clip(rel, 0, bw - 1), axis=1)
            t = jnp.where((rel >= 0) & (rel < bw), g, jnp.float32(0.0))
            acc = t if acc is None else acc + t
        cols.append(acc)
    return jnp.concatenate(cols, axis=1) if nb_out > 1 else cols[0]


def _lane_gather_lerp(x, d, idx, frac, ranges, bw, boundary=True, neg=False):
    rows, src_w = x.shape
    _, out_w = idx.shape
    nb_out = max(1, out_w // bw)
    cols = []
    for o in range(nb_out):
        if nb_out > 1:
            io = idx[:, o * bw:(o + 1) * bw]
            fo = frac[:, o * bw:(o + 1) * bw]
        else:
            io, fo = idx, frac
        acc = None
        lo, hi = ranges[o]
        for s in range(lo, hi + 1):
            rel = io - s * bw
            crel = jnp.clip(rel, 0, bw - 1)
            gx = jnp.take_along_axis(x[:, s * bw:(s + 1) * bw], crel, axis=1)
            gd = jnp.take_along_axis(d[:, s * bw:(s + 1) * bw], crel, axis=1)
            if neg and s == o:
                inb = rel >= 0
            else:
                inb = (rel >= 0) & (rel < bw)
            t = jnp.where(inb, gx + fo * gd, jnp.float32(0.0))
            acc = t if acc is None else acc + t
        if boundary:
            c0 = lo * bw - 1
            if c0 < 0:
                bval = fo * x[:, 0:1]
            else:
                bval = x[:, c0:c0 + 1] + fo * d[:, c0:c0 + 1]
            acc = acc + jnp.where(io == c0, bval, jnp.float32(0.0))
        cols.append(acc)
    return jnp.concatenate(cols, axis=1) if nb_out > 1 else cols[0]


def _pool9(x):
    hs = x + _shl(x, 1) + _shl(x, 2)
    vs = hs + _ru(hs, 1) + _ru(hs, 2)
    return vs * jnp.float32(1.0 / 9.0)


def _scale_kernel(il_ref, ir_ref, disp_ref, rc_ref, xw_ref, out_ref,
                  sl_ref, sbl_ref, sr_ref, sbr_ref,
                  *, s, h, w, f, big_w, R, K, bw, hranges0):
    nb = max(1, w // bw)

    w0 = rc_ref[:, 0:1]
    w1 = rc_ref[:, 1:2]
    w2 = rc_ref[:, 2:3]

    def blend(x):
        return w0 * _rd(x, 1) + w1 * x + w2 * _ru(x, 1)

    if s > 0:
        x0h = xw_ref[1:2, :].astype(jnp.int32)
        wxh = xw_ref[3:4, :]

    def resized(ref, c):
        if s == 0:
            return ref[0, c]
        v = None
        for k in range(f):
            pk = ref[0, c, :, k, :]
            term = rc_ref[:, 3 + k:4 + k] * pk
            v = term if v is None else v + term
            if k == 0:
                p0 = pk
        v = v + rc_ref[:, 3 + f:4 + f] * _ru(p0, 1)
        dv = _shl(v, 1) - v
        i0 = jnp.broadcast_to(x0h, (h, w))
        f0 = jnp.broadcast_to(wxh, (h, w))
        return _lane_gather_lerp(v, dv, i0, f0, hranges0, bw, boundary=False)

    for c in range(3):
        o_l = resized(il_ref, c)
        sl_ref[c * h:(c + 1) * h, :] = o_l
        sbl_ref[c * h:(c + 1) * h, :] = blend(o_l)
        o_r = resized(ir_ref, c)
        sr_ref[c * h:(c + 1) * h, :] = o_r
        sbr_ref[c * h:(c + 1) * h, :] = blend(o_r)
    dl = -disp_ref[0, 0]
    dr = disp_ref[0, 1]
    sl_ref[3 * h:4 * h, :] = dr
    sbl_ref[3 * h:4 * h, :] = blend(dr)
    sr_ref[3 * h:4 * h, :] = dl
    sbr_ref[3 * h:4 * h, :] = blend(dl)
    zpad = jnp.zeros((8, w), jnp.float32)
    sl_ref[4 * h:4 * h + 8, :] = zpad
    sbl_ref[4 * h:4 * h + 8, :] = zpad
    sr_ref[4 * h:4 * h + 8, :] = zpad
    sbr_ref[4 * h:4 * h + 8, :] = zpad

    xb = xw_ref[0:1, :]
    iota_k = jax.lax.broadcasted_iota(jnp.int32, (K, 1), 0)
    lane_i = jax.lax.broadcasted_iota(jnp.int32, (1, w), 1)
    lm2 = lane_i < (w - 2)
    lm1f = (lane_i < (w - 1)).astype(jnp.float32)
    lane8 = jax.lax.broadcasted_iota(jnp.int32, (1, _LANES), 1)

    inv_wm1 = jnp.float32(1.0 / (w - 1))
    inv_w = jnp.float32(1.0 / w)

    def chunk(cidx, acc):
        r0 = pl.multiple_of(cidx * R, 8)

        def load4(ref):
            return [ref[pl.ds(c * h + r0, K), :] for c in range(4)]

        gl = load4(sl_ref)
        gr = load4(sr_ref)
        bl = jnp.concatenate(load4(sbl_ref), axis=0)
        br = jnp.concatenate(load4(sbr_ref), axis=0)
        drw = gl[3]
        dlw = gr[3]

        def warp_idx(d):
            gx = jnp.float32(2.0) * (xb + d) - jnp.float32(1.0)
            ix = ((gx + jnp.float32(1.0)) * jnp.float32(w) - jnp.float32(1.0)) * jnp.float32(0.5)
            x0f = jnp.floor(ix)
            return x0f.astype(jnp.int32), ix - x0f

        x0l, fxl = warp_idx(dlw)
        x0r, fxr = warp_idx(drw)

        def warp(src4, x0, fx, kind):
            i4 = jnp.concatenate([x0] * 4, axis=0)
            f4 = jnp.concatenate([fx] * 4, axis=0)
            d4 = _shl(src4, 1) - src4
            return _lane_gather_lerp(src4, d4, i4, f4, _dyn_ranges(kind, nb), bw,
                                     neg=(kind == "neg"))

        est_l = warp(bl, x0l, fxl, "neg")
        est_r = warp(br, x0r, fxr, "pos")

        m_ssim = (iota_k < R) & ((iota_k + r0) < (h - 2))
        m3 = jnp.concatenate([m_ssim] * 3, axis=0) & lm2

        def ssim_sum(gt3, e3):
            mu_g = _pool9(gt3)
            mu_e = _pool9(e3)
            sig_g = _pool9(gt3 * gt3) - mu_g * mu_g
            sig_e = _pool9(e3 * e3) - mu_e * mu_e
            sig_ge = _pool9(gt3 * e3) - mu_e * mu_g
            num = (jnp.float32(2.0) * mu_e * mu_g + _C1) * (jnp.float32(2.0) * sig_ge + _C2)
            den = (mu_g * mu_g + mu_e * mu_e + _C1) * (sig_g + sig_e + _C2)
            ssim = num / den
            return jnp.sum(jnp.where(m3, ssim, jnp.float32(0.0)))

        s_ssim_l = ssim_sum(jnp.concatenate(gl[:3], axis=0), est_l[0:3 * K])
        s_ssim_r = ssim_sum(jnp.concatenate(gr[:3], axis=0), est_r[0:3 * K])

        m_row = (iota_k < R).astype(jnp.float32)
        m_dy = ((iota_k < R) & ((iota_k + r0) < (h - 1))).astype(jnp.float32)

        def smooth_sums(dw, img3):
            sdx = jnp.sum(jnp.abs(dw - _shl(dw, 1)) * lm1f, axis=1, keepdims=True)
            sdy = jnp.sum(jnp.abs(dw - _ru(dw, 1)), axis=1, keepdims=True)
            swx = None
            swy = None
            for c in range(3):
                rx = jnp.sum(jnp.abs(img3[c] - _shl(img3[c], 1)) * lm1f,
                             axis=1, keepdims=True)
                ry = jnp.sum(jnp.abs(img3[c] - _ru(img3[c], 1)), axis=1, keepdims=True)
                ex = jnp.exp(-rx * inv_wm1)
                ey = jnp.exp(-ry * inv_w)
                swx = ex if swx is None else swx + ex
                swy = ey if swy is None else swy + ey
            sx = jnp.sum(sdx * swx * m_row)
            sy = jnp.sum(sdy * swy * m_dy)
            return sx, sy

        s_xl, s_yl = smooth_sums(dlw, gl)
        s_xr, s_yr = smooth_sums(drw, gr)

        mrow2 = m_row
        s_lrc_r = jnp.sum(jnp.abs(drw - est_l[3 * K:4 * K]) * mrow2)
        s_lrc_l = jnp.sum(jnp.abs(dlw - est_r[3 * K:4 * K]) * mrow2)

        vals = [s_ssim_l, s_ssim_r, s_xl, s_yl, s_xr, s_yr, s_lrc_r, s_lrc_l]
        upd = acc
        for k, v in enumerate(vals):
            upd = upd + jnp.where(lane8 == k, v, jnp.float32(0.0))
        return upd

    acc = jax.lax.fori_loop(0, h // R, chunk, jnp.zeros((1, _LANES), jnp.float32))
    out_ref[0] = acc


def _scale_call(s, il, ir, disp):
    B, _, H, W = il.shape
    h, w, f = H >> s, W >> s, 1 << s
    R = min(64, h)
    K = R + 8
    bw = min(w, _LANES)
    nb = max(1, w // bw)

    rc = np.zeros((h, 16), np.float32)
    rc[:, 0:3] = _warp_blend_taps(h)
    if s > 0:
        rc[:, 3:3 + f + 1] = _resize_vert_taps(H, h, f)

    xw = np.zeros((8, w), np.float32)
    xw[0] = _np_linspace_f32(0.0, 1.0, w)
    hranges0 = ((0, 0),)
    if s > 0:
        x0h, x1h, wxh = _resize_horiz_idx(W, w)
        xw[1] = x0h.astype(np.float32)
        xw[2] = x1h.astype(np.float32)
        xw[3] = wxh
        nb_src = W // bw
        hranges0 = tuple(_static_ranges(x0h, nb, bw, nb_src))
        il_in = il.reshape(B, 3, h, f, W)
        ir_in = ir.reshape(B, 3, h, f, W)
        img_block = (1, 3, h, f, W)
        img_map = lambda b: (b, 0, 0, 0, 0)
    else:
        il_in, ir_in = il, ir
        img_block = (1, 3, H, W)
        img_map = lambda b: (b, 0, 0, 0)

    body = functools.partial(
        _scale_kernel, s=s, h=h, w=w, f=f, big_w=W, R=R, K=K, bw=bw,
        hranges0=hranges0)

    out = pl.pallas_call(
        body,
        grid=(B,),
        in_specs=[
            pl.BlockSpec(img_block, img_map),
            pl.BlockSpec(img_block, img_map),
            pl.BlockSpec((1, 2, h, w), lambda b: (b, 0, 0, 0)),
            pl.BlockSpec((h, 16), lambda b: (0, 0)),
            pl.BlockSpec((8, w), lambda b: (0, 0)),
        ],
        out_specs=pl.BlockSpec((1, 1, _LANES), lambda b: (b, 0, 0)),
        out_shape=jax.ShapeDtypeStruct((B, 1, _LANES), jnp.float32),
        scratch_shapes=[pltpu.VMEM((4 * h + 8, w), jnp.float32)
                        for _ in range(4)],
        compiler_params=pltpu.CompilerParams(
            dimension_semantics=("parallel",)),
    )(il_in, ir_in, disp, jnp.asarray(rc), jnp.asarray(xw))
    return jnp.sum(out[:, 0, :8], axis=0)


def kernel(images_left, images_right, disp0, disp1, disp2, disp3):
    B, _, H, W = images_left.shape
    disps = [disp0, disp1, disp2, disp3]
    apps, smooths, lrcs = [], [], []
    for s in range(4):
        h, w = H >> s, W >> s
        S = _scale_call(s, images_left, images_right, disps[s])
        n_ssim = np.float32(B * 3 * (h - 2) * (w - 2))
        n_x = np.float32(B * 3 * h * (w - 1))
        n_y = np.float32(B * 3 * (h - 1) * w)
        n_d = np.float32(B * h * w)
        apps.append((S[0] + S[1]) / n_ssim)
        smooths.append(((S[2] + S[4]) / n_x + (S[3] + S[5]) / n_y)
                       / np.float32((s + 1) ** 2))
        lrcs.append((S[6] + S[7]) / n_d)
    return jnp.stack([
        jnp.mean(jnp.stack(apps)),
        jnp.mean(jnp.stack(smooths)),
        jnp.mean(jnp.stack(lrcs)),
    ])


# speedup vs baseline: 1.0915x; 1.0198x over previous
import functools

import numpy as np
import jax
import jax.numpy as jnp
from jax.experimental import pallas as pl
from jax.experimental.pallas import tpu as pltpu

_C1 = np.float32(0.0001)
_C2 = np.float32(0.0005)
_LANES = 128


def _f32(x):
    return np.asarray(x, np.float32)


def _np_linspace_f32(a, b, n):
    return np.linspace(a, b, n).astype(np.float32)


def _warp_blend_taps(h):
    yb = _np_linspace_f32(0.0, 1.0, h)
    gy = (np.float32(2.0) * yb - np.float32(1.0)).astype(np.float32)
    iy = ((gy + np.float32(1.0)) * np.float32(h) - np.float32(1.0)) * np.float32(0.5)
    y0 = np.floor(iy).astype(np.int64)
    wy = (iy - y0.astype(np.float32)).astype(np.float32)
    taps = np.zeros((h, 3), np.float32)
    for r in range(h):
        for yy, ww in ((y0[r], np.float32(1.0) - wy[r]), (y0[r] + 1, wy[r])):
            if 0 <= yy < h:
                t = int(yy - r + 1)
                assert 0 <= t <= 2
                taps[r, t] += ww
    return taps


def _resize_vert_taps(n_in, n_out, f):
    ys = _np_linspace_f32(0.0, n_in - 1.0, n_out)
    y0 = np.floor(ys).astype(np.int64)
    y1 = np.minimum(y0 + 1, n_in - 1)
    wy = (ys - y0.astype(np.float32)).astype(np.float32)
    taps = np.zeros((n_out, f + 1), np.float32)
    for i in range(n_out):
        for yy, ww in ((y0[i], np.float32(1.0) - wy[i]), (y1[i], wy[i])):
            t = int(yy - f * i)
            assert 0 <= t <= f, (i, yy, f)
            taps[i, t] += ww
    return taps


def _resize_horiz_idx(n_in, n_out):
    xs = _np_linspace_f32(0.0, n_in - 1.0, n_out)
    x0 = np.floor(xs).astype(np.int64)
    x1 = np.minimum(x0 + 1, n_in - 1)
    wx = (xs - x0.astype(np.float32)).astype(np.float32)
    return x0.astype(np.int32), x1.astype(np.int32), wx


def _static_ranges(idx, nb_out, bw, nb_src):
    out = []
    for o in range(nb_out):
        seg = idx[o * bw:(o + 1) * bw]
        out.append((max(0, int(seg.min()) // bw), min(nb_src - 1, int(seg.max()) // bw)))
    return out


def _dyn_ranges(kind, nb):
    out = []
    for o in range(nb):
        if kind == "neg":
            lo, hi = 0, o
        else:
            lo, hi = o, nb - 1
        out.append((lo, hi))
    return out


def _shl(x, k):
    r, c = x.shape
    return jnp.concatenate([x[:, k:], jnp.zeros((r, k), jnp.float32)], axis=1)


def _shu(x, k=1):
    r, c = x.shape
    return jnp.concatenate([x[k:, :], jnp.zeros((k, c), jnp.float32)], axis=0)


def _shd(x, k=1):
    r, c = x.shape
    return jnp.concatenate([jnp.zeros((k, c), jnp.float32), x[:-k, :]], axis=0)


def _rl(x, k):
    return pltpu.roll(x, x.shape[1] - k, 1)


def _ru(x, k):
    return pltpu.roll(x, x.shape[0] - k, 0)


def _rd(x, k):
    return pltpu.roll(x, k, 0)


def _lane_gather(x, idx, ranges, bw):
    rows, src_w = x.shape
    _, out_w = idx.shape
    nb_out = max(1, out_w // bw)
    cols = []
    for o in range(nb_out):
        if nb_out > 1:
            io = idx[:, o * bw:(o + 1) * bw]
        else:
            io = idx
        acc = None
        lo, hi = ranges[o]
        for s in range(lo, hi + 1):
            src = x[:, s * bw:(s + 1) * bw]
            rel = io - s * bw
            g = jnp.take_along_axis(src, jnp.clip(rel, 0, bw - 1), axis=1)
            t = jnp.where((rel >= 0) & (rel < bw), g, jnp.float32(0.0))
            acc = t if acc is None else acc + t
        cols.append(acc)
    return jnp.concatenate(cols, axis=1) if nb_out > 1 else cols[0]


def _lane_gather_lerp(x, d, idx, frac, ranges, bw, boundary=True, neg=False):
    rows, src_w = x.shape
    _, out_w = idx.shape
    nb_out = max(1, out_w // bw)
    cols = []
    for o in range(nb_out):
        if nb_out > 1:
            io = idx[:, o * bw:(o + 1) * bw]
            fo = frac[:, o * bw:(o + 1) * bw]
        else:
            io, fo = idx, frac
        acc = None
        lo, hi = ranges[o]
        for s in range(lo, hi + 1):
            rel = io - s * bw
            crel = jnp.clip(rel, 0, bw - 1)
            gx = jnp.take_along_axis(x[:, s * bw:(s + 1) * bw], crel, axis=1)
            gd = jnp.take_along_axis(d[:, s * bw:(s + 1) * bw], crel, axis=1)
            if neg and s == o:
                inb = rel >= 0
            else:
                inb = (rel >= 0) & (rel < bw)
            t = jnp.where(inb, gx + fo * gd, jnp.float32(0.0))
            acc = t if acc is None else acc + t
        if boundary:
            c0 = lo * bw - 1
            if c0 < 0:
                bval = fo * x[:, 0:1]
            else:
                bval = x[:, c0:c0 + 1] + fo * d[:, c0:c0 + 1]
            acc = acc + jnp.where(io == c0, bval, jnp.float32(0.0))
        cols.append(acc)
    return jnp.concatenate(cols, axis=1) if nb_out > 1 else cols[0]


def _pool9(x):
    hs = x + _shl(x, 1) + _shl(x, 2)
    vs = hs + _ru(hs, 1) + _ru(hs, 2)
    return vs * jnp.float32(1.0 / 9.0)


def _scale_kernel(il_ref, ir_ref, disp_ref, rc_ref, xw_ref, out_ref,
                  sl_ref, sbl_ref, sr_ref, sbr_ref,
                  *, s, h, w, f, big_w, R, K, bw, hranges0):
    nb = max(1, w // bw)

    w0 = rc_ref[:, 0:1]
    w1 = rc_ref[:, 1:2]
    w2 = rc_ref[:, 2:3]

    def blend(x):
        return w0 * _rd(x, 1) + w1 * x + w2 * _ru(x, 1)

    if s > 0:
        x0h = xw_ref[1:2, :].astype(jnp.int32)
        wxh = xw_ref[3:4, :]

    def resized(ref, c):
        if s == 0:
            return ref[0, c]
        v = None
        for k in range(f):
            pk = ref[0, c, :, k, :]
            term = rc_ref[:, 3 + k:4 + k] * pk
            v = term if v is None else v + term
            if k == 0:
                p0 = pk
        v = v + rc_ref[:, 3 + f:4 + f] * _ru(p0, 1)
        dv = _shl(v, 1) - v
        i0 = jnp.broadcast_to(x0h, (h, w))
        f0 = jnp.broadcast_to(wxh, (h, w))
        return _lane_gather_lerp(v, dv, i0, f0, hranges0, bw, boundary=False)

    for c in range(3):
        o_l = resized(il_ref, c)
        sl_ref[c * h:(c + 1) * h, :] = o_l
        sbl_ref[c * h:(c + 1) * h, :] = blend(o_l)
        o_r = resized(ir_ref, c)
        sr_ref[c * h:(c + 1) * h, :] = o_r
        sbr_ref[c * h:(c + 1) * h, :] = blend(o_r)
    dl = -disp_ref[0, 0]
    dr = disp_ref[0, 1]
    sl_ref[3 * h:4 * h, :] = dr
    sbl_ref[3 * h:4 * h, :] = blend(dr)
    sr_ref[3 * h:4 * h, :] = dl
    sbr_ref[3 * h:4 * h, :] = blend(dl)
    zpad = jnp.zeros((8, w), jnp.float32)
    sl_ref[4 * h:4 * h + 8, :] = zpad
    sbl_ref[4 * h:4 * h + 8, :] = zpad
    sr_ref[4 * h:4 * h + 8, :] = zpad
    sbr_ref[4 * h:4 * h + 8, :] = zpad

    xb = xw_ref[0:1, :]
    iota_k = jax.lax.broadcasted_iota(jnp.int32, (K, 1), 0)
    lane_i = jax.lax.broadcasted_iota(jnp.int32, (1, w), 1)
    lm2 = lane_i < (w - 2)
    lm1f = (lane_i < (w - 1)).astype(jnp.float32)
    lane8 = jax.lax.broadcasted_iota(jnp.int32, (1, _LANES), 1)

    inv_wm1 = jnp.float32(1.0 / (w - 1))
    inv_w = jnp.float32(1.0 / w)

    def chunk(cidx, acc):
        r0 = pl.multiple_of(cidx * R, 8)

        def load4(ref):
            return [ref[pl.ds(c * h + r0, K), :] for c in range(4)]

        gl = load4(sl_ref)
        gr = load4(sr_ref)
        bl = jnp.concatenate(load4(sbl_ref), axis=0)
        br = jnp.concatenate(load4(sbr_ref), axis=0)
        drw = gl[3]
        dlw = gr[3]

        def warp_idx(d):
            gx = jnp.float32(2.0) * (xb + d) - jnp.float32(1.0)
            ix = ((gx + jnp.float32(1.0)) * jnp.float32(w) - jnp.float32(1.0)) * jnp.float32(0.5)
            x0f = jnp.floor(ix)
            return x0f.astype(jnp.int32), ix - x0f

        x0l, fxl = warp_idx(dlw)
        x0r, fxr = warp_idx(drw)

        def warp(src4, x0, fx, kind):
            i4 = jnp.concatenate([x0] * 4, axis=0)
            f4 = jnp.concatenate([fx] * 4, axis=0)
            d4 = _shl(src4, 1) - src4
            return _lane_gather_lerp(src4, d4, i4, f4, _dyn_ranges(kind, nb), bw,
                                     neg=(kind == "neg"))

        est_l = warp(bl, x0l, fxl, "neg")
        est_r = warp(br, x0r, fxr, "pos")

        m_ssim = (iota_k < R) & ((iota_k + r0) < (h - 2))
        m3 = jnp.concatenate([m_ssim] * 3, axis=0) & lm2

        def ssim_sum(gt3, e3):
            mu_g = _pool9(gt3)
            mu_e = _pool9(e3)
            sig_g = _pool9(gt3 * gt3) - mu_g * mu_g
            sig_e = _pool9(e3 * e3) - mu_e * mu_e
            sig_ge = _pool9(gt3 * e3) - mu_e * mu_g
            num = (jnp.float32(2.0) * mu_e * mu_g + _C1) * (jnp.float32(2.0) * sig_ge + _C2)
            den = (mu_g * mu_g + mu_e * mu_e + _C1) * (sig_g + sig_e + _C2)
            ssim = num / den
            return jnp.sum(jnp.where(m3, ssim, jnp.float32(0.0)))

        s_ssim_l = ssim_sum(jnp.concatenate(gl[:3], axis=0), est_l[0:3 * K])
        s_ssim_r = ssim_sum(jnp.concatenate(gr[:3], axis=0), est_r[0:3 * K])

        m_row = (iota_k < R).astype(jnp.float32)
        m_dy = ((iota_k < R) & ((iota_k + r0) < (h - 1))).astype(jnp.float32)

        def smooth_sums(dw, img3):
            sdx = jnp.sum(jnp.abs(dw - _shl(dw, 1)) * lm1f, axis=1, keepdims=True)
            sdy = jnp.sum(jnp.abs(dw - _ru(dw, 1)), axis=1, keepdims=True)
            swx = None
            swy = None
            for c in range(3):
                rx = jnp.sum(jnp.abs(img3[c] - _shl(img3[c], 1)) * lm1f,
                             axis=1, keepdims=True)
                ry = jnp.sum(jnp.abs(img3[c] - _ru(img3[c], 1)), axis=1, keepdims=True)
                ex = jnp.exp(-rx * inv_wm1)
                ey = jnp.exp(-ry * inv_w)
                swx = ex if swx is None else swx + ex
                swy = ey if swy is None else swy + ey
            sx = jnp.sum(sdx * swx * m_row)
            sy = jnp.sum(sdy * swy * m_dy)
            return sx, sy

        s_xl, s_yl = smooth_sums(dlw, gl)
        s_xr, s_yr = smooth_sums(drw, gr)

        mrow2 = m_row
        s_lrc_r = jnp.sum(jnp.abs(drw - est_l[3 * K:4 * K]) * mrow2)
        s_lrc_l = jnp.sum(jnp.abs(dlw - est_r[3 * K:4 * K]) * mrow2)

        vals = [s_ssim_l, s_ssim_r, s_xl, s_yl, s_xr, s_yr, s_lrc_r, s_lrc_l]
        upd = acc
        for k, v in enumerate(vals):
            upd = upd + jnp.where(lane8 == k, v, jnp.float32(0.0))
        return upd

    acc = jax.lax.fori_loop(0, h // R, chunk, jnp.zeros((1, _LANES), jnp.float32))
    out_ref[0] = acc


def _scale_call(s, il, ir, disp):
    B, _, H, W = il.shape
    h, w, f = H >> s, W >> s, 1 << s
    R = min(128, h)
    K = R + 8
    bw = min(w, _LANES)
    nb = max(1, w // bw)

    rc = np.zeros((h, 16), np.float32)
    rc[:, 0:3] = _warp_blend_taps(h)
    if s > 0:
        rc[:, 3:3 + f + 1] = _resize_vert_taps(H, h, f)

    xw = np.zeros((8, w), np.float32)
    xw[0] = _np_linspace_f32(0.0, 1.0, w)
    hranges0 = ((0, 0),)
    if s > 0:
        x0h, x1h, wxh = _resize_horiz_idx(W, w)
        xw[1] = x0h.astype(np.float32)
        xw[2] = x1h.astype(np.float32)
        xw[3] = wxh
        nb_src = W // bw
        hranges0 = tuple(_static_ranges(x0h, nb, bw, nb_src))
        il_in = il.reshape(B, 3, h, f, W)
        ir_in = ir.reshape(B, 3, h, f, W)
        img_block = (1, 3, h, f, W)
        img_map = lambda b: (b, 0, 0, 0, 0)
    else:
        il_in, ir_in = il, ir
        img_block = (1, 3, H, W)
        img_map = lambda b: (b, 0, 0, 0)

    body = functools.partial(
        _scale_kernel, s=s, h=h, w=w, f=f, big_w=W, R=R, K=K, bw=bw,
        hranges0=hranges0)

    out = pl.pallas_call(
        body,
        grid=(B,),
        in_specs=[
            pl.BlockSpec(img_block, img_map),
            pl.BlockSpec(img_block, img_map),
            pl.BlockSpec((1, 2, h, w), lambda b: (b, 0, 0, 0)),
            pl.BlockSpec((h, 16), lambda b: (0, 0)),
            pl.BlockSpec((8, w), lambda b: (0, 0)),
        ],
        out_specs=pl.BlockSpec((1, 1, _LANES), lambda b: (b, 0, 0)),
        out_shape=jax.ShapeDtypeStruct((B, 1, _LANES), jnp.float32),
        scratch_shapes=[pltpu.VMEM((4 * h + 8, w), jnp.float32)
                        for _ in range(4)],
        compiler_params=pltpu.CompilerParams(
            dimension_semantics=("parallel",)),
    )(il_in, ir_in, disp, jnp.asarray(rc), jnp.asarray(xw))
    return jnp.sum(out[:, 0, :8], axis=0)


def kernel(images_left, images_right, disp0, disp1, disp2, disp3):
    B, _, H, W = images_left.shape
    disps = [disp0, disp1, disp2, disp3]
    apps, smooths, lrcs = [], [], []
    for s in range(4):
        h, w = H >> s, W >> s
        S = _scale_call(s, images_left, images_right, disps[s])
        n_ssim = np.float32(B * 3 * (h - 2) * (w - 2))
        n_x = np.float32(B * 3 * h * (w - 1))
        n_y = np.float32(B * 3 * (h - 1) * w)
        n_d = np.float32(B * h * w)
        apps.append((S[0] + S[1]) / n_ssim)
        smooths.append(((S[2] + S[4]) / n_x + (S[3] + S[5]) / n_y)
                       / np.float32((s + 1) ** 2))
        lrcs.append((S[6] + S[7]) / n_d)
    return jnp.stack([
        jnp.mean(jnp.stack(apps)),
        jnp.mean(jnp.stack(smooths)),
        jnp.mean(jnp.stack(lrcs)),
    ])


# speedup vs baseline: 1.1203x; 1.0264x over previous
import functools

import numpy as np
import jax
import jax.numpy as jnp
from jax.experimental import pallas as pl
from jax.experimental.pallas import tpu as pltpu

_C1 = np.float32(0.0001)
_C2 = np.float32(0.0005)
_LANES = 128


def _f32(x):
    return np.asarray(x, np.float32)


def _np_linspace_f32(a, b, n):
    return np.linspace(a, b, n).astype(np.float32)


def _warp_blend_taps(h):
    yb = _np_linspace_f32(0.0, 1.0, h)
    gy = (np.float32(2.0) * yb - np.float32(1.0)).astype(np.float32)
    iy = ((gy + np.float32(1.0)) * np.float32(h) - np.float32(1.0)) * np.float32(0.5)
    y0 = np.floor(iy).astype(np.int64)
    wy = (iy - y0.astype(np.float32)).astype(np.float32)
    taps = np.zeros((h, 3), np.float32)
    for r in range(h):
        for yy, ww in ((y0[r], np.float32(1.0) - wy[r]), (y0[r] + 1, wy[r])):
            if 0 <= yy < h:
                t = int(yy - r + 1)
                assert 0 <= t <= 2
                taps[r, t] += ww
    return taps


def _resize_vert_taps(n_in, n_out, f):
    ys = _np_linspace_f32(0.0, n_in - 1.0, n_out)
    y0 = np.floor(ys).astype(np.int64)
    y1 = np.minimum(y0 + 1, n_in - 1)
    wy = (ys - y0.astype(np.float32)).astype(np.float32)
    taps = np.zeros((n_out, f + 1), np.float32)
    for i in range(n_out):
        for yy, ww in ((y0[i], np.float32(1.0) - wy[i]), (y1[i], wy[i])):
            t = int(yy - f * i)
            assert 0 <= t <= f, (i, yy, f)
            taps[i, t] += ww
    return taps


def _resize_horiz_idx(n_in, n_out):
    xs = _np_linspace_f32(0.0, n_in - 1.0, n_out)
    x0 = np.floor(xs).astype(np.int64)
    x1 = np.minimum(x0 + 1, n_in - 1)
    wx = (xs - x0.astype(np.float32)).astype(np.float32)
    return x0.astype(np.int32), x1.astype(np.int32), wx


def _static_ranges(idx, nb_out, bw, nb_src):
    out = []
    for o in range(nb_out):
        seg = idx[o * bw:(o + 1) * bw]
        out.append((max(0, int(seg.min()) // bw), min(nb_src - 1, int(seg.max()) // bw)))
    return out


def _dyn_ranges(kind, nb):
    out = []
    for o in range(nb):
        if kind == "neg":
            lo, hi = 0, o
        else:
            lo, hi = o, nb - 1
        out.append((lo, hi))
    return out


def _shl(x, k):
    r, c = x.shape
    return jnp.concatenate([x[:, k:], jnp.zeros((r, k), jnp.float32)], axis=1)


def _shu(x, k=1):
    r, c = x.shape
    return jnp.concatenate([x[k:, :], jnp.zeros((k, c), jnp.float32)], axis=0)


def _shd(x, k=1):
    r, c = x.shape
    return jnp.concatenate([jnp.zeros((k, c), jnp.float32), x[:-k, :]], axis=0)


def _rl(x, k):
    return pltpu.roll(x, x.shape[1] - k, 1)


def _ru(x, k):
    return pltpu.roll(x, x.shape[0] - k, 0)


def _rd(x, k):
    return pltpu.roll(x, k, 0)


def _lane_gather(x, idx, ranges, bw):
    rows, src_w = x.shape
    _, out_w = idx.shape
    nb_out = max(1, out_w // bw)
    cols = []
    for o in range(nb_out):
        if nb_out > 1:
            io = idx[:, o * bw:(o + 1) * bw]
        else:
            io = idx
        acc = None
        lo, hi = ranges[o]
        for s in range(lo, hi + 1):
            src = x[:, s * bw:(s + 1) * bw]
            rel = io - s * bw
            g = jnp.take_along_axis(src, jnp.clip(rel, 0, bw - 1), axis=1)
            t = jnp.where((rel >= 0) & (rel < bw), g, jnp.float32(0.0))
            acc = t if acc is None else acc + t
        cols.append(acc)
    return jnp.concatenate(cols, axis=1) if nb_out > 1 else cols[0]


def _lane_gather_lerp(x, d, idx, frac, ranges, bw, boundary=True, neg=False):
    rows, src_w = x.shape
    _, out_w = idx.shape
    nb_out = max(1, out_w // bw)
    cols = []
    for o in range(nb_out):
        if nb_out > 1:
            io = idx[:, o * bw:(o + 1) * bw]
            fo = frac[:, o * bw:(o + 1) * bw]
        else:
            io, fo = idx, frac
        acc = None
        lo, hi = ranges[o]
        for s in range(lo, hi + 1):
            rel = io - s * bw
            crel = jnp.clip(rel, 0, bw - 1)
            gx = jnp.take_along_axis(x[:, s * bw:(s + 1) * bw], crel, axis=1)
            gd = jnp.take_along_axis(d[:, s * bw:(s + 1) * bw], crel, axis=1)
            if neg and s == o:
                inb = rel >= 0
            else:
                inb = (rel >= 0) & (rel < bw)
            t = jnp.where(inb, gx + fo * gd, jnp.float32(0.0))
            acc = t if acc is None else acc + t
        if boundary:
            c0 = lo * bw - 1
            if c0 < 0:
                bval = fo * x[:, 0:1]
            else:
                bval = x[:, c0:c0 + 1] + fo * d[:, c0:c0 + 1]
            acc = acc + jnp.where(io == c0, bval, jnp.float32(0.0))
        cols.append(acc)
    return jnp.concatenate(cols, axis=1) if nb_out > 1 else cols[0]


def _pool9(x):
    hs = x + _shl(x, 1) + _shl(x, 2)
    vs = hs + _ru(hs, 1) + _ru(hs, 2)
    return vs * jnp.float32(1.0 / 9.0)


def _scale_kernel(il_ref, ir_ref, disp_ref, rc_ref, xw_ref, out_ref,
                  *scratch,
                  s, h, w, f, R, K, bw, hranges0, chunks):
    nb = max(1, w // bw)

    w0 = rc_ref[:, 0:1]
    w1 = rc_ref[:, 1:2]
    w2 = rc_ref[:, 2:3]

    def blend(x):
        return w0 * _rd(x, 1) + w1 * x + w2 * _ru(x, 1)

    if s > 0:
        x0h = xw_ref[1:2, :].astype(jnp.int32)
        wxh = xw_ref[3:4, :]

    def resized(ref, c):
        v = None
        for k in range(f):
            pk = ref[0, c, :, k, :]
            term = rc_ref[:, 3 + k:4 + k] * pk
            v = term if v is None else v + term
            if k == 0:
                p0 = pk
        v = v + rc_ref[:, 3 + f:4 + f] * _ru(p0, 1)
        dv = _shl(v, 1) - v
        i0 = jnp.broadcast_to(x0h, (h, w))
        f0 = jnp.broadcast_to(wxh, (h, w))
        return _lane_gather_lerp(v, dv, i0, f0, hranges0, bw, boundary=False)

    xb = xw_ref[0:1, :]
    iota_k = jax.lax.broadcasted_iota(jnp.int32, (K, 1), 0)
    lane_i = jax.lax.broadcasted_iota(jnp.int32, (1, w), 1)
    lm2 = lane_i < (w - 2)
    lm1f = (lane_i < (w - 1)).astype(jnp.float32)
    lane8 = jax.lax.broadcasted_iota(jnp.int32, (1, _LANES), 1)

    inv_wm1 = jnp.float32(1.0 / (w - 1))
    inv_w = jnp.float32(1.0 / w)

    def chunk_body(acc, gl, gr, drw, dlw, bl, br, loff, lstart):
        def warp_idx(d):
            gx = jnp.float32(2.0) * (xb + d) - jnp.float32(1.0)
            ix = ((gx + jnp.float32(1.0)) * jnp.float32(w) - jnp.float32(1.0)) * jnp.float32(0.5)
            x0f = jnp.floor(ix)
            return x0f.astype(jnp.int32), ix - x0f

        x0l, fxl = warp_idx(dlw)
        x0r, fxr = warp_idx(drw)

        def warp(src4, x0, fx, kind):
            i4 = jnp.concatenate([x0] * 4, axis=0)
            f4 = jnp.concatenate([fx] * 4, axis=0)
            d4 = _shl(src4, 1) - src4
            return _lane_gather_lerp(src4, d4, i4, f4, _dyn_ranges(kind, nb), bw,
                                     neg=(kind == "neg"))

        est_l = warp(bl, x0l, fxl, "neg")
        est_r = warp(br, x0r, fxr, "pos")

        m_ssim = ((iota_k >= lstart) & (iota_k < lstart + R)
                  & ((iota_k + loff) < (h - 2)))
        m3 = jnp.concatenate([m_ssim] * 3, axis=0) & lm2

        def ssim_sum(gt3, e3):
            mu_g = _pool9(gt3)
            mu_e = _pool9(e3)
            sig_g = _pool9(gt3 * gt3) - mu_g * mu_g
            sig_e = _pool9(e3 * e3) - mu_e * mu_e
            sig_ge = _pool9(gt3 * e3) - mu_e * mu_g
            num = (jnp.float32(2.0) * mu_e * mu_g + _C1) * (jnp.float32(2.0) * sig_ge + _C2)
            den = (mu_g * mu_g + mu_e * mu_e + _C1) * (sig_g + sig_e + _C2)
            ssim = num / den
            return jnp.sum(jnp.where(m3, ssim, jnp.float32(0.0)))

        s_ssim_l = ssim_sum(jnp.concatenate(gl, axis=0), est_l[0:3 * K])
        s_ssim_r = ssim_sum(jnp.concatenate(gr, axis=0), est_r[0:3 * K])

        m_row = ((iota_k >= lstart) & (iota_k < lstart + R)).astype(jnp.float32)
        m_dy = ((iota_k >= lstart) & (iota_k < lstart + R)
                & ((iota_k + loff) < (h - 1))).astype(jnp.float32)

        def smooth_sums(dw, img3):
            sdx = jnp.sum(jnp.abs(dw - _shl(dw, 1)) * lm1f, axis=1, keepdims=True)
            sdy = jnp.sum(jnp.abs(dw - _ru(dw, 1)), axis=1, keepdims=True)
            swx = None
            swy = None
            for c in range(3):
                rx = jnp.sum(jnp.abs(img3[c] - _shl(img3[c], 1)) * lm1f,
                             axis=1, keepdims=True)
                ry = jnp.sum(jnp.abs(img3[c] - _ru(img3[c], 1)), axis=1, keepdims=True)
                ex = jnp.exp(-rx * inv_wm1)
                ey = jnp.exp(-ry * inv_w)
                swx = ex if swx is None else swx + ex
                swy = ey if swy is None else swy + ey
            sx = jnp.sum(sdx * swx * m_row)
            sy = jnp.sum(sdy * swy * m_dy)
            return sx, sy

        s_xl, s_yl = smooth_sums(dlw, gl)
        s_xr, s_yr = smooth_sums(drw, gr)

        s_lrc_r = jnp.sum(jnp.abs(drw - est_l[3 * K:4 * K]) * m_row)
        s_lrc_l = jnp.sum(jnp.abs(dlw - est_r[3 * K:4 * K]) * m_row)

        vals = [s_ssim_l, s_ssim_r, s_xl, s_yl, s_xr, s_yr, s_lrc_r, s_lrc_l]
        for k, v in enumerate(vals):
            acc = acc + jnp.where(lane8 == k, v, jnp.float32(0.0))
        return acc

    acc = jnp.zeros((1, _LANES), jnp.float32)
    if s == 0:
        sbl_ref, sbr_ref = scratch
        for c in range(3):
            sbl_ref[c * h:(c + 1) * h, :] = blend(il_ref[0, c])
            sbr_ref[c * h:(c + 1) * h, :] = blend(ir_ref[0, c])
        sbl_ref[3 * h:4 * h, :] = blend(disp_ref[0, 1])
        sbr_ref[3 * h:4 * h, :] = blend(-disp_ref[0, 0])
        for loff, lstart in chunks:
            gl = [il_ref[0, c, loff:loff + K, :] for c in range(3)]
            gr = [ir_ref[0, c, loff:loff + K, :] for c in range(3)]
            drw = disp_ref[0, 1, loff:loff + K, :]
            dlw = -disp_ref[0, 0, loff:loff + K, :]
            bl = jnp.concatenate(
                [sbl_ref[c * h + loff:c * h + loff + K, :] for c in range(4)], axis=0)
            br = jnp.concatenate(
                [sbr_ref[c * h + loff:c * h + loff + K, :] for c in range(4)], axis=0)
            acc = chunk_body(acc, gl, gr, drw, dlw, bl, br, loff, lstart)
    else:
        ils = [resized(il_ref, c) for c in range(3)]
        irs = [resized(ir_ref, c) for c in range(3)]
        dlw = -disp_ref[0, 0]
        drw = disp_ref[0, 1]
        bl = jnp.concatenate([blend(x) for x in ils] + [blend(drw)], axis=0)
        br = jnp.concatenate([blend(x) for x in irs] + [blend(dlw)], axis=0)
        acc = chunk_body(acc, ils, irs, drw, dlw, bl, br, 0, 0)
    out_ref[0] = acc


def _scale_call(s, il, ir, disp):
    B, _, H, W = il.shape
    h, w, f = H >> s, W >> s, 1 << s
    if h > 128:
        R, K = 128, 136
        chunks = tuple((min(i * R, h - K), i * R - min(i * R, h - K))
                       for i in range(h // R))
    else:
        R, K = h, h
        chunks = ((0, 0),)
    bw = min(w, _LANES)
    nb = max(1, w // bw)

    rc = np.zeros((h, 16), np.float32)
    rc[:, 0:3] = _warp_blend_taps(h)
    if s > 0:
        rc[:, 3:3 + f + 1] = _resize_vert_taps(H, h, f)

    xw = np.zeros((8, w), np.float32)
    xw[0] = _np_linspace_f32(0.0, 1.0, w)
    hranges0 = ((0, 0),)
    if s > 0:
        x0h, x1h, wxh = _resize_horiz_idx(W, w)
        xw[1] = x0h.astype(np.float32)
        xw[2] = x1h.astype(np.float32)
        xw[3] = wxh
        nb_src = W // bw
        hranges0 = tuple(_static_ranges(x0h, nb, bw, nb_src))
        il_in = il.reshape(B, 3, h, f, W)
        ir_in = ir.reshape(B, 3, h, f, W)
        img_block = (1, 3, h, f, W)
        img_map = lambda b: (b, 0, 0, 0, 0)
    else:
        il_in, ir_in = il, ir
        img_block = (1, 3, H, W)
        img_map = lambda b: (b, 0, 0, 0)

    body = functools.partial(
        _scale_kernel, s=s, h=h, w=w, f=f, R=R, K=K, bw=bw,
        hranges0=hranges0, chunks=chunks)

    out = pl.pallas_call(
        body,
        grid=(B,),
        in_specs=[
            pl.BlockSpec(img_block, img_map),
            pl.BlockSpec(img_block, img_map),
            pl.BlockSpec((1, 2, h, w), lambda b: (b, 0, 0, 0)),
            pl.BlockSpec((h, 16), lambda b: (0, 0)),
            pl.BlockSpec((8, w), lambda b: (0, 0)),
        ],
        out_specs=pl.BlockSpec((1, 1, _LANES), lambda b: (b, 0, 0)),
        out_shape=jax.ShapeDtypeStruct((B, 1, _LANES), jnp.float32),
        scratch_shapes=([pltpu.VMEM((4 * h, w), jnp.float32)
                         for _ in range(2)] if s == 0 else []),
        compiler_params=pltpu.CompilerParams(
            dimension_semantics=("parallel",)),
    )(il_in, ir_in, disp, jnp.asarray(rc), jnp.asarray(xw))
    return jnp.sum(out[:, 0, :8], axis=0)


def kernel(images_left, images_right, disp0, disp1, disp2, disp3):
    B, _, H, W = images_left.shape
    disps = [disp0, disp1, disp2, disp3]
    apps, smooths, lrcs = [], [], []
    for s in range(4):
        h, w = H >> s, W >> s
        S = _scale_call(s, images_left, images_right, disps[s])
        n_ssim = np.float32(B * 3 * (h - 2) * (w - 2))
        n_x = np.float32(B * 3 * h * (w - 1))
        n_y = np.float32(B * 3 * (h - 1) * w)
        n_d = np.float32(B * h * w)
        apps.append((S[0] + S[1]) / n_ssim)
        smooths.append(((S[2] + S[4]) / n_x + (S[3] + S[5]) / n_y)
                       / np.float32((s + 1) ** 2))
        lrcs.append((S[6] + S[7]) / n_d)
    return jnp.stack([
        jnp.mean(jnp.stack(apps)),
        jnp.mean(jnp.stack(smooths)),
        jnp.mean(jnp.stack(lrcs)),
    ])


# speedup vs baseline: 1.1229x; 1.0023x over previous
import functools

import numpy as np
import jax
import jax.numpy as jnp
from jax.experimental import pallas as pl
from jax.experimental.pallas import tpu as pltpu

_C1 = np.float32(0.0001)
_C2 = np.float32(0.0005)
_LANES = 128


def _f32(x):
    return np.asarray(x, np.float32)


def _np_linspace_f32(a, b, n):
    return np.linspace(a, b, n).astype(np.float32)


def _warp_blend_taps(h):
    yb = _np_linspace_f32(0.0, 1.0, h)
    gy = (np.float32(2.0) * yb - np.float32(1.0)).astype(np.float32)
    iy = ((gy + np.float32(1.0)) * np.float32(h) - np.float32(1.0)) * np.float32(0.5)
    y0 = np.floor(iy).astype(np.int64)
    wy = (iy - y0.astype(np.float32)).astype(np.float32)
    taps = np.zeros((h, 3), np.float32)
    for r in range(h):
        for yy, ww in ((y0[r], np.float32(1.0) - wy[r]), (y0[r] + 1, wy[r])):
            if 0 <= yy < h:
                t = int(yy - r + 1)
                assert 0 <= t <= 2
                taps[r, t] += ww
    return taps


def _resize_vert_taps(n_in, n_out, f):
    ys = _np_linspace_f32(0.0, n_in - 1.0, n_out)
    y0 = np.floor(ys).astype(np.int64)
    y1 = np.minimum(y0 + 1, n_in - 1)
    wy = (ys - y0.astype(np.float32)).astype(np.float32)
    taps = np.zeros((n_out, f + 1), np.float32)
    for i in range(n_out):
        for yy, ww in ((y0[i], np.float32(1.0) - wy[i]), (y1[i], wy[i])):
            t = int(yy - f * i)
            assert 0 <= t <= f, (i, yy, f)
            taps[i, t] += ww
    return taps


def _resize_horiz_idx(n_in, n_out):
    xs = _np_linspace_f32(0.0, n_in - 1.0, n_out)
    x0 = np.floor(xs).astype(np.int64)
    x1 = np.minimum(x0 + 1, n_in - 1)
    wx = (xs - x0.astype(np.float32)).astype(np.float32)
    return x0.astype(np.int32), x1.astype(np.int32), wx


def _static_ranges(idx, nb_out, bw, nb_src):
    out = []
    for o in range(nb_out):
        seg = idx[o * bw:(o + 1) * bw]
        out.append((max(0, int(seg.min()) // bw), min(nb_src - 1, int(seg.max()) // bw)))
    return out


def _dyn_ranges(kind, nb):
    out = []
    for o in range(nb):
        if kind == "neg":
            lo, hi = 0, o
        else:
            lo, hi = o, nb - 1
        out.append((lo, hi))
    return out


def _shl(x, k):
    r, c = x.shape
    return jnp.concatenate([x[:, k:], jnp.zeros((r, k), jnp.float32)], axis=1)


def _shu(x, k=1):
    r, c = x.shape
    return jnp.concatenate([x[k:, :], jnp.zeros((k, c), jnp.float32)], axis=0)


def _shd(x, k=1):
    r, c = x.shape
    return jnp.concatenate([jnp.zeros((k, c), jnp.float32), x[:-k, :]], axis=0)


def _rl(x, k):
    return pltpu.roll(x, x.shape[1] - k, 1)


def _ru(x, k):
    return pltpu.roll(x, x.shape[0] - k, 0)


def _rd(x, k):
    return pltpu.roll(x, k, 0)


def _lane_gather(x, idx, ranges, bw):
    rows, src_w = x.shape
    _, out_w = idx.shape
    nb_out = max(1, out_w // bw)
    cols = []
    for o in range(nb_out):
        if nb_out > 1:
            io = idx[:, o * bw:(o + 1) * bw]
        else:
            io = idx
        acc = None
        lo, hi = ranges[o]
        for s in range(lo, hi + 1):
            src = x[:, s * bw:(s + 1) * bw]
            rel = io - s * bw
            g = jnp.take_along_axis(src, jnp.clip(rel, 0, bw - 1), axis=1)
            t = jnp.where((rel >= 0) & (rel < bw), g, jnp.float32(0.0))
            acc = t if acc is None else acc + t
        cols.append(acc)
    return jnp.concatenate(cols, axis=1) if nb_out > 1 else cols[0]


def _lane_gather_lerp_multi(srcs, idx, frac, ranges, bw, neg=False):
    _, out_w = idx.shape
    nb_out = max(1, out_w // bw)
    n = len(srcs)
    cols = [[] for _ in range(n)]
    for o in range(nb_out):
        if nb_out > 1:
            io = idx[:, o * bw:(o + 1) * bw]
            fo = frac[:, o * bw:(o + 1) * bw]
        else:
            io, fo = idx, frac
        accs = [None] * n
        lo, hi = ranges[o]
        for s in range(lo, hi + 1):
            rel = io - s * bw
            crel = jnp.clip(rel, 0, bw - 1)
            if neg and s == o:
                inb = rel >= 0
            else:
                inb = (rel >= 0) & (rel < bw)
            for i, (x, d) in enumerate(srcs):
                gx = jnp.take_along_axis(x[:, s * bw:(s + 1) * bw], crel, axis=1)
                gd = jnp.take_along_axis(d[:, s * bw:(s + 1) * bw], crel, axis=1)
                t = jnp.where(inb, gx + fo * gd, jnp.float32(0.0))
                accs[i] = t if accs[i] is None else accs[i] + t
        c0 = lo * bw - 1
        bm = io == c0
        for i, (x, d) in enumerate(srcs):
            if c0 < 0:
                bval = fo * x[:, 0:1]
            else:
                bval = x[:, c0:c0 + 1] + fo * d[:, c0:c0 + 1]
            cols[i].append(accs[i] + jnp.where(bm, bval, jnp.float32(0.0)))
    if nb_out > 1:
        return [jnp.concatenate(c, axis=1) for c in cols]
    return [c[0] for c in cols]


def _lane_gather_lerp(x, d, idx, frac, ranges, bw, boundary=True, neg=False):
    rows, src_w = x.shape
    _, out_w = idx.shape
    nb_out = max(1, out_w // bw)
    cols = []
    for o in range(nb_out):
        if nb_out > 1:
            io = idx[:, o * bw:(o + 1) * bw]
            fo = frac[:, o * bw:(o + 1) * bw]
        else:
            io, fo = idx, frac
        acc = None
        lo, hi = ranges[o]
        for s in range(lo, hi + 1):
            rel = io - s * bw
            crel = jnp.clip(rel, 0, bw - 1)
            gx = jnp.take_along_axis(x[:, s * bw:(s + 1) * bw], crel, axis=1)
            gd = jnp.take_along_axis(d[:, s * bw:(s + 1) * bw], crel, axis=1)
            if neg and s == o:
                inb = rel >= 0
            else:
                inb = (rel >= 0) & (rel < bw)
            t = jnp.where(inb, gx + fo * gd, jnp.float32(0.0))
            acc = t if acc is None else acc + t
        if boundary:
            c0 = lo * bw - 1
            if c0 < 0:
                bval = fo * x[:, 0:1]
            else:
                bval = x[:, c0:c0 + 1] + fo * d[:, c0:c0 + 1]
            acc = acc + jnp.where(io == c0, bval, jnp.float32(0.0))
        cols.append(acc)
    return jnp.concatenate(cols, axis=1) if nb_out > 1 else cols[0]


def _pool9(x):
    hs = x + _shl(x, 1) + _shl(x, 2)
    vs = hs + _ru(hs, 1) + _ru(hs, 2)
    return vs * jnp.float32(1.0 / 9.0)


def _scale_kernel(il_ref, ir_ref, disp_ref, rc_ref, xw_ref, out_ref,
                  *, s, h, w, f, R, K, bw, hranges0, chunks):
    nb = max(1, w // bw)

    if s > 0:
        x0h = xw_ref[1:2, :].astype(jnp.int32)
        wxh = xw_ref[3:4, :]

    def resized(ref, c):
        v = None
        for k in range(f):
            pk = ref[0, c, :, k, :]
            term = rc_ref[:, 3 + k:4 + k] * pk
            v = term if v is None else v + term
            if k == 0:
                p0 = pk
        v = v + rc_ref[:, 3 + f:4 + f] * _ru(p0, 1)
        dv = _shl(v, 1) - v
        i0 = jnp.broadcast_to(x0h, (h, w))
        f0 = jnp.broadcast_to(wxh, (h, w))
        return _lane_gather_lerp(v, dv, i0, f0, hranges0, bw, boundary=False)

    xb = xw_ref[0:1, :]
    iota_k = jax.lax.broadcasted_iota(jnp.int32, (K, 1), 0)
    lane_i = jax.lax.broadcasted_iota(jnp.int32, (1, w), 1)
    lm2 = lane_i < (w - 2)
    lm1f = (lane_i < (w - 1)).astype(jnp.float32)
    lane8 = jax.lax.broadcasted_iota(jnp.int32, (1, _LANES), 1)

    inv_wm1 = jnp.float32(1.0 / (w - 1))
    inv_w = jnp.float32(1.0 / w)

    def chunk_body(acc, gl, gr, drw, dlw, loff, lstart):
        w0 = rc_ref[loff:loff + K, 0:1]
        w1 = rc_ref[loff:loff + K, 1:2]
        w2 = rc_ref[loff:loff + K, 2:3]

        def blend(x):
            return w0 * _rd(x, 1) + w1 * x + w2 * _ru(x, 1)

        def warp_idx(d):
            gx = jnp.float32(2.0) * (xb + d) - jnp.float32(1.0)
            ix = ((gx + jnp.float32(1.0)) * jnp.float32(w) - jnp.float32(1.0)) * jnp.float32(0.5)
            x0f = jnp.floor(ix)
            return x0f.astype(jnp.int32), ix - x0f

        x0l, fxl = warp_idx(dlw)
        x0r, fxr = warp_idx(drw)

        def warp(raw4, x0, fx, kind):
            srcs = []
            for x in raw4:
                b = blend(x)
                srcs.append((b, _shl(b, 1) - b))
            return _lane_gather_lerp_multi(srcs, x0, fx, _dyn_ranges(kind, nb),
                                           bw, neg=(kind == "neg"))

        est_l = warp(gl + [drw], x0l, fxl, "neg")
        est_r = warp(gr + [dlw], x0r, fxr, "pos")

        m_ssim = ((iota_k >= lstart) & (iota_k < lstart + R)
                  & ((iota_k + loff) < (h - 2)))
        m3 = jnp.concatenate([m_ssim] * 3, axis=0) & lm2

        def ssim_sum(gt3, e3):
            mu_g = _pool9(gt3)
            mu_e = _pool9(e3)
            sig_g = _pool9(gt3 * gt3) - mu_g * mu_g
            sig_e = _pool9(e3 * e3) - mu_e * mu_e
            sig_ge = _pool9(gt3 * e3) - mu_e * mu_g
            num = (jnp.float32(2.0) * mu_e * mu_g + _C1) * (jnp.float32(2.0) * sig_ge + _C2)
            den = (mu_g * mu_g + mu_e * mu_e + _C1) * (sig_g + sig_e + _C2)
            ssim = num / den
            return jnp.sum(jnp.where(m3, ssim, jnp.float32(0.0)))

        s_ssim_l = ssim_sum(jnp.concatenate(gl, axis=0),
                            jnp.concatenate(est_l[:3], axis=0))
        s_ssim_r = ssim_sum(jnp.concatenate(gr, axis=0),
                            jnp.concatenate(est_r[:3], axis=0))

        m_row = ((iota_k >= lstart) & (iota_k < lstart + R)).astype(jnp.float32)
        m_dy = ((iota_k >= lstart) & (iota_k < lstart + R)
                & ((iota_k + loff) < (h - 1))).astype(jnp.float32)

        def smooth_sums(dw, img3):
            sdx = jnp.sum(jnp.abs(dw - _shl(dw, 1)) * lm1f, axis=1, keepdims=True)
            sdy = jnp.sum(jnp.abs(dw - _ru(dw, 1)), axis=1, keepdims=True)
            swx = None
            swy = None
            for c in range(3):
                rx = jnp.sum(jnp.abs(img3[c] - _shl(img3[c], 1)) * lm1f,
                             axis=1, keepdims=True)
                ry = jnp.sum(jnp.abs(img3[c] - _ru(img3[c], 1)), axis=1, keepdims=True)
                ex = jnp.exp(-rx * inv_wm1)
                ey = jnp.exp(-ry * inv_w)
                swx = ex if swx is None else swx + ex
                swy = ey if swy is None else swy + ey
            sx = jnp.sum(sdx * swx * m_row)
            sy = jnp.sum(sdy * swy * m_dy)
            return sx, sy

        s_xl, s_yl = smooth_sums(dlw, gl)
        s_xr, s_yr = smooth_sums(drw, gr)

        s_lrc_r = jnp.sum(jnp.abs(drw - est_l[3]) * m_row)
        s_lrc_l = jnp.sum(jnp.abs(dlw - est_r[3]) * m_row)

        vals = [s_ssim_l, s_ssim_r, s_xl, s_yl, s_xr, s_yr, s_lrc_r, s_lrc_l]
        for k, v in enumerate(vals):
            acc = acc + jnp.where(lane8 == k, v, jnp.float32(0.0))
        return acc

    acc = jnp.zeros((1, _LANES), jnp.float32)
    if s == 0:
        for loff, lstart in chunks:
            gl = [il_ref[0, c, loff:loff + K, :] for c in range(3)]
            gr = [ir_ref[0, c, loff:loff + K, :] for c in range(3)]
            drw = disp_ref[0, 1, loff:loff + K, :]
            dlw = -disp_ref[0, 0, loff:loff + K, :]
            acc = chunk_body(acc, gl, gr, drw, dlw, loff, lstart)
    else:
        ils = [resized(il_ref, c) for c in range(3)]
        irs = [resized(ir_ref, c) for c in range(3)]
        dlw = -disp_ref[0, 0]
        drw = disp_ref[0, 1]
        acc = chunk_body(acc, ils, irs, drw, dlw, 0, 0)
    out_ref[0] = acc


def _scale_call(s, il, ir, disp):
    B, _, H, W = il.shape
    h, w, f = H >> s, W >> s, 1 << s
    if h > 128:
        R, K = 128, 136
        chunks = tuple((min(i * R, h - K), i * R - min(i * R, h - K))
                       for i in range(h // R))
    else:
        R, K = h, h
        chunks = ((0, 0),)
    bw = min(w, _LANES)
    nb = max(1, w // bw)

    rc = np.zeros((h, 16), np.float32)
    rc[:, 0:3] = _warp_blend_taps(h)
    if s > 0:
        rc[:, 3:3 + f + 1] = _resize_vert_taps(H, h, f)

    xw = np.zeros((8, w), np.float32)
    xw[0] = _np_linspace_f32(0.0, 1.0, w)
    hranges0 = ((0, 0),)
    if s > 0:
        x0h, x1h, wxh = _resize_horiz_idx(W, w)
        xw[1] = x0h.astype(np.float32)
        xw[2] = x1h.astype(np.float32)
        xw[3] = wxh
        nb_src = W // bw
        hranges0 = tuple(_static_ranges(x0h, nb, bw, nb_src))
        il_in = il.reshape(B, 3, h, f, W)
        ir_in = ir.reshape(B, 3, h, f, W)
        img_block = (1, 3, h, f, W)
        img_map = lambda b: (b, 0, 0, 0, 0)
    else:
        il_in, ir_in = il, ir
        img_block = (1, 3, H, W)
        img_map = lambda b: (b, 0, 0, 0)

    body = functools.partial(
        _scale_kernel, s=s, h=h, w=w, f=f, R=R, K=K, bw=bw,
        hranges0=hranges0, chunks=chunks)

    out = pl.pallas_call(
        body,
        grid=(B,),
        in_specs=[
            pl.BlockSpec(img_block, img_map),
            pl.BlockSpec(img_block, img_map),
            pl.BlockSpec((1, 2, h, w), lambda b: (b, 0, 0, 0)),
            pl.BlockSpec((h, 16), lambda b: (0, 0)),
            pl.BlockSpec((8, w), lambda b: (0, 0)),
        ],
        out_specs=pl.BlockSpec((1, 1, _LANES), lambda b: (b, 0, 0)),
        out_shape=jax.ShapeDtypeStruct((B, 1, _LANES), jnp.float32),
        compiler_params=pltpu.CompilerParams(
            dimension_semantics=("parallel",)),
    )(il_in, ir_in, disp, jnp.asarray(rc), jnp.asarray(xw))
    return jnp.sum(out[:, 0, :8], axis=0)


def kernel(images_left, images_right, disp0, disp1, disp2, disp3):
    B, _, H, W = images_left.shape
    disps = [disp0, disp1, disp2, disp3]
    apps, smooths, lrcs = [], [], []
    for s in range(4):
        h, w = H >> s, W >> s
        S = _scale_call(s, images_left, images_right, disps[s])
        n_ssim = np.float32(B * 3 * (h - 2) * (w - 2))
        n_x = np.float32(B * 3 * h * (w - 1))
        n_y = np.float32(B * 3 * (h - 1) * w)
        n_d = np.float32(B * h * w)
        apps.append((S[0] + S[1]) / n_ssim)
        smooths.append(((S[2] + S[4]) / n_x + (S[3] + S[5]) / n_y)
                       / np.float32((s + 1) ** 2))
        lrcs.append((S[6] + S[7]) / n_d)
    return jnp.stack([
        jnp.mean(jnp.stack(apps)),
        jnp.mean(jnp.stack(smooths)),
        jnp.mean(jnp.stack(lrcs)),
    ])


# speedup vs baseline: 1.1278x; 1.0043x over previous
import functools

import numpy as np
import jax
import jax.numpy as jnp
from jax.experimental import pallas as pl
from jax.experimental.pallas import tpu as pltpu

_C1 = np.float32(0.0001)
_C2 = np.float32(0.0005)
_LANES = 128


def _f32(x):
    return np.asarray(x, np.float32)


def _np_linspace_f32(a, b, n):
    return np.linspace(a, b, n).astype(np.float32)


def _warp_blend_taps(h):
    yb = _np_linspace_f32(0.0, 1.0, h)
    gy = (np.float32(2.0) * yb - np.float32(1.0)).astype(np.float32)
    iy = ((gy + np.float32(1.0)) * np.float32(h) - np.float32(1.0)) * np.float32(0.5)
    y0 = np.floor(iy).astype(np.int64)
    wy = (iy - y0.astype(np.float32)).astype(np.float32)
    taps = np.zeros((h, 3), np.float32)
    for r in range(h):
        for yy, ww in ((y0[r], np.float32(1.0) - wy[r]), (y0[r] + 1, wy[r])):
            if 0 <= yy < h:
                t = int(yy - r + 1)
                assert 0 <= t <= 2
                taps[r, t] += ww
    return taps


def _resize_vert_taps(n_in, n_out, f):
    ys = _np_linspace_f32(0.0, n_in - 1.0, n_out)
    y0 = np.floor(ys).astype(np.int64)
    y1 = np.minimum(y0 + 1, n_in - 1)
    wy = (ys - y0.astype(np.float32)).astype(np.float32)
    taps = np.zeros((n_out, f + 1), np.float32)
    for i in range(n_out):
        for yy, ww in ((y0[i], np.float32(1.0) - wy[i]), (y1[i], wy[i])):
            t = int(yy - f * i)
            assert 0 <= t <= f, (i, yy, f)
            taps[i, t] += ww
    return taps


def _resize_horiz_idx(n_in, n_out):
    xs = _np_linspace_f32(0.0, n_in - 1.0, n_out)
    x0 = np.floor(xs).astype(np.int64)
    x1 = np.minimum(x0 + 1, n_in - 1)
    wx = (xs - x0.astype(np.float32)).astype(np.float32)
    return x0.astype(np.int32), x1.astype(np.int32), wx


def _static_ranges(idx, nb_out, bw, nb_src):
    out = []
    for o in range(nb_out):
        seg = idx[o * bw:(o + 1) * bw]
        out.append((max(0, int(seg.min()) // bw), min(nb_src - 1, int(seg.max()) // bw)))
    return out


def _dyn_ranges(kind, nb):
    out = []
    for o in range(nb):
        if kind == "neg":
            lo, hi = 0, o
        else:
            lo, hi = o, nb - 1
        out.append((lo, hi))
    return out


def _shl(x, k):
    r, c = x.shape
    return jnp.concatenate([x[:, k:], jnp.zeros((r, k), jnp.float32)], axis=1)


def _shu(x, k=1):
    r, c = x.shape
    return jnp.concatenate([x[k:, :], jnp.zeros((k, c), jnp.float32)], axis=0)


def _shd(x, k=1):
    r, c = x.shape
    return jnp.concatenate([jnp.zeros((k, c), jnp.float32), x[:-k, :]], axis=0)


def _rl(x, k):
    return pltpu.roll(x, x.shape[1] - k, 1)


def _ru(x, k):
    return pltpu.roll(x, x.shape[0] - k, 0)


def _rd(x, k):
    return pltpu.roll(x, k, 0)


def _lane_gather(x, idx, ranges, bw):
    rows, src_w = x.shape
    _, out_w = idx.shape
    nb_out = max(1, out_w // bw)
    cols = []
    for o in range(nb_out):
        if nb_out > 1:
            io = idx[:, o * bw:(o + 1) * bw]
        else:
            io = idx
        acc = None
        lo, hi = ranges[o]
        for s in range(lo, hi + 1):
            src = x[:, s * bw:(s + 1) * bw]
            rel = io - s * bw
            g = jnp.take_along_axis(src, jnp.clip(rel, 0, bw - 1), axis=1)
            t = jnp.where((rel >= 0) & (rel < bw), g, jnp.float32(0.0))
            acc = t if acc is None else acc + t
        cols.append(acc)
    return jnp.concatenate(cols, axis=1) if nb_out > 1 else cols[0]


def _lane_gather_lerp_multi(srcs, idx, frac, ranges, bw, neg=False):
    _, out_w = idx.shape
    nb_out = max(1, out_w // bw)
    n = len(srcs)
    cols = [[] for _ in range(n)]
    for o in range(nb_out):
        if nb_out > 1:
            io = idx[:, o * bw:(o + 1) * bw]
            fo = frac[:, o * bw:(o + 1) * bw]
        else:
            io, fo = idx, frac
        accs = [None] * n
        lo, hi = ranges[o]
        for s in range(lo, hi + 1):
            rel = io - s * bw
            crel = jnp.clip(rel, 0, bw - 1)
            if neg and s == o:
                inb = rel >= 0
            else:
                inb = jax.lax.bitcast_convert_type(rel, jnp.uint32) < jnp.uint32(bw)
            for i, (x, d) in enumerate(srcs):
                gx = jnp.take_along_axis(x[:, s * bw:(s + 1) * bw], crel, axis=1)
                gd = jnp.take_along_axis(d[:, s * bw:(s + 1) * bw], crel, axis=1)
                t = jnp.where(inb, gx + fo * gd, jnp.float32(0.0))
                accs[i] = t if accs[i] is None else accs[i] + t
        c0 = lo * bw - 1
        bm = io == c0
        for i, (x, d) in enumerate(srcs):
            if c0 < 0:
                bval = fo * x[:, 0:1]
            else:
                bval = x[:, c0:c0 + 1] + fo * d[:, c0:c0 + 1]
            cols[i].append(accs[i] + jnp.where(bm, bval, jnp.float32(0.0)))
    if nb_out > 1:
        return [jnp.concatenate(c, axis=1) for c in cols]
    return [c[0] for c in cols]


def _lane_gather_lerp(x, d, idx, frac, ranges, bw, boundary=True, neg=False):
    rows, src_w = x.shape
    _, out_w = idx.shape
    nb_out = max(1, out_w // bw)
    cols = []
    for o in range(nb_out):
        if nb_out > 1:
            io = idx[:, o * bw:(o + 1) * bw]
            fo = frac[:, o * bw:(o + 1) * bw]
        else:
            io, fo = idx, frac
        acc = None
        lo, hi = ranges[o]
        for s in range(lo, hi + 1):
            rel = io - s * bw
            crel = jnp.clip(rel, 0, bw - 1)
            gx = jnp.take_along_axis(x[:, s * bw:(s + 1) * bw], crel, axis=1)
            gd = jnp.take_along_axis(d[:, s * bw:(s + 1) * bw], crel, axis=1)
            if neg and s == o:
                inb = rel >= 0
            else:
                inb = (rel >= 0) & (rel < bw)
            t = jnp.where(inb, gx + fo * gd, jnp.float32(0.0))
            acc = t if acc is None else acc + t
        if boundary:
            c0 = lo * bw - 1
            if c0 < 0:
                bval = fo * x[:, 0:1]
            else:
                bval = x[:, c0:c0 + 1] + fo * d[:, c0:c0 + 1]
            acc = acc + jnp.where(io == c0, bval, jnp.float32(0.0))
        cols.append(acc)
    return jnp.concatenate(cols, axis=1) if nb_out > 1 else cols[0]


def _pool9(x):
    hs = x + _shl(x, 1) + _shl(x, 2)
    vs = hs + _ru(hs, 1) + _ru(hs, 2)
    return vs * jnp.float32(1.0 / 9.0)


def _scale_kernel(il_ref, ir_ref, disp_ref, rc_ref, xw_ref, out_ref,
                  *, s, h, w, f, R, K, bw, hranges0, chunks):
    nb = max(1, w // bw)

    if s > 0:
        x0h = xw_ref[1:2, :].astype(jnp.int32)
        wxh = xw_ref[3:4, :]

    def resized(ref, c):
        v = None
        for k in range(f):
            pk = ref[0, c, :, k, :]
            term = rc_ref[:, 3 + k:4 + k] * pk
            v = term if v is None else v + term
            if k == 0:
                p0 = pk
        v = v + rc_ref[:, 3 + f:4 + f] * _ru(p0, 1)
        dv = _shl(v, 1) - v
        i0 = jnp.broadcast_to(x0h, (h, w))
        f0 = jnp.broadcast_to(wxh, (h, w))
        return _lane_gather_lerp(v, dv, i0, f0, hranges0, bw, boundary=False)

    xb = xw_ref[0:1, :]
    iota_k = jax.lax.broadcasted_iota(jnp.int32, (K, 1), 0)
    lane_i = jax.lax.broadcasted_iota(jnp.int32, (1, w), 1)
    lm2 = lane_i < (w - 2)
    lm1f = (lane_i < (w - 1)).astype(jnp.float32)
    lane8 = jax.lax.broadcasted_iota(jnp.int32, (1, _LANES), 1)

    inv_wm1 = jnp.float32(1.0 / (w - 1))
    inv_w = jnp.float32(1.0 / w)

    def chunk_body(acc, gl, gr, drw, dlw, loff, lstart):
        w0 = rc_ref[loff:loff + K, 0:1]
        w1 = rc_ref[loff:loff + K, 1:2]
        w2 = rc_ref[loff:loff + K, 2:3]

        def blend(x):
            return w0 * _rd(x, 1) + w1 * x + w2 * _ru(x, 1)

        def warp_idx(d):
            gx = jnp.float32(2.0) * (xb + d) - jnp.float32(1.0)
            ix = ((gx + jnp.float32(1.0)) * jnp.float32(w) - jnp.float32(1.0)) * jnp.float32(0.5)
            x0f = jnp.floor(ix)
            return x0f.astype(jnp.int32), ix - x0f

        x0l, fxl = warp_idx(dlw)
        x0r, fxr = warp_idx(drw)

        def warp(raw4, x0, fx, kind):
            srcs = []
            for x in raw4:
                b = blend(x)
                srcs.append((b, _shl(b, 1) - b))
            return _lane_gather_lerp_multi(srcs, x0, fx, _dyn_ranges(kind, nb),
                                           bw, neg=(kind == "neg"))

        est_l = warp(gl + [drw], x0l, fxl, "neg")
        est_r = warp(gr + [dlw], x0r, fxr, "pos")

        m_ssim = ((iota_k >= lstart) & (iota_k < lstart + R)
                  & ((iota_k + loff) < (h - 2)))
        m3 = jnp.concatenate([m_ssim] * 3, axis=0) & lm2

        def ssim_sum(gt3, e3):
            mu_g = _pool9(gt3)
            mu_e = _pool9(e3)
            sig_g = _pool9(gt3 * gt3) - mu_g * mu_g
            sig_e = _pool9(e3 * e3) - mu_e * mu_e
            sig_ge = _pool9(gt3 * e3) - mu_e * mu_g
            num = (jnp.float32(2.0) * mu_e * mu_g + _C1) * (jnp.float32(2.0) * sig_ge + _C2)
            den = (mu_g * mu_g + mu_e * mu_e + _C1) * (sig_g + sig_e + _C2)
            ssim = num / den
            return jnp.sum(jnp.where(m3, ssim, jnp.float32(0.0)))

        s_ssim_l = ssim_sum(jnp.concatenate(gl, axis=0),
                            jnp.concatenate(est_l[:3], axis=0))
        s_ssim_r = ssim_sum(jnp.concatenate(gr, axis=0),
                            jnp.concatenate(est_r[:3], axis=0))

        m_row = ((iota_k >= lstart) & (iota_k < lstart + R)).astype(jnp.float32)
        m_dy = ((iota_k >= lstart) & (iota_k < lstart + R)
                & ((iota_k + loff) < (h - 1))).astype(jnp.float32)

        def smooth_sums(dw, img3):
            sdx = jnp.sum(jnp.abs(dw - _shl(dw, 1)) * lm1f, axis=1, keepdims=True)
            sdy = jnp.sum(jnp.abs(dw - _ru(dw, 1)), axis=1, keepdims=True)
            swx = None
            swy = None
            for c in range(3):
                rx = jnp.sum(jnp.abs(img3[c] - _shl(img3[c], 1)) * lm1f,
                             axis=1, keepdims=True)
                ry = jnp.sum(jnp.abs(img3[c] - _ru(img3[c], 1)), axis=1, keepdims=True)
                ex = jnp.exp(-rx * inv_wm1)
                ey = jnp.exp(-ry * inv_w)
                swx = ex if swx is None else swx + ex
                swy = ey if swy is None else swy + ey
            sx = jnp.sum(sdx * swx * m_row)
            sy = jnp.sum(sdy * swy * m_dy)
            return sx, sy

        s_xl, s_yl = smooth_sums(dlw, gl)
        s_xr, s_yr = smooth_sums(drw, gr)

        s_lrc_r = jnp.sum(jnp.abs(drw - est_l[3]) * m_row)
        s_lrc_l = jnp.sum(jnp.abs(dlw - est_r[3]) * m_row)

        vals = [s_ssim_l, s_ssim_r, s_xl, s_yl, s_xr, s_yr, s_lrc_r, s_lrc_l]
        for k, v in enumerate(vals):
            acc = acc + jnp.where(lane8 == k, v, jnp.float32(0.0))
        return acc

    acc = jnp.zeros((1, _LANES), jnp.float32)
    if s == 0:
        for loff, lstart in chunks:
            gl = [il_ref[0, c, loff:loff + K, :] for c in range(3)]
            gr = [ir_ref[0, c, loff:loff + K, :] for c in range(3)]
            drw = disp_ref[0, 1, loff:loff + K, :]
            dlw = -disp_ref[0, 0, loff:loff + K, :]
            acc = chunk_body(acc, gl, gr, drw, dlw, loff, lstart)
    else:
        ils = [resized(il_ref, c) for c in range(3)]
        irs = [resized(ir_ref, c) for c in range(3)]
        dlw = -disp_ref[0, 0]
        drw = disp_ref[0, 1]
        acc = chunk_body(acc, ils, irs, drw, dlw, 0, 0)
    out_ref[0] = acc


def _scale_call(s, il, ir, disp):
    B, _, H, W = il.shape
    h, w, f = H >> s, W >> s, 1 << s
    if h > 128:
        R, K = 128, 136
        chunks = tuple((min(i * R, h - K), i * R - min(i * R, h - K))
                       for i in range(h // R))
    else:
        R, K = h, h
        chunks = ((0, 0),)
    bw = min(w, _LANES)
    nb = max(1, w // bw)

    rc = np.zeros((h, 16), np.float32)
    rc[:, 0:3] = _warp_blend_taps(h)
    if s > 0:
        rc[:, 3:3 + f + 1] = _resize_vert_taps(H, h, f)

    xw = np.zeros((8, w), np.float32)
    xw[0] = _np_linspace_f32(0.0, 1.0, w)
    hranges0 = ((0, 0),)
    if s > 0:
        x0h, x1h, wxh = _resize_horiz_idx(W, w)
        xw[1] = x0h.astype(np.float32)
        xw[2] = x1h.astype(np.float32)
        xw[3] = wxh
        nb_src = W // bw
        hranges0 = tuple(_static_ranges(x0h, nb, bw, nb_src))
        il_in = il.reshape(B, 3, h, f, W)
        ir_in = ir.reshape(B, 3, h, f, W)
        img_block = (1, 3, h, f, W)
        img_map = lambda b: (b, 0, 0, 0, 0)
    else:
        il_in, ir_in = il, ir
        img_block = (1, 3, H, W)
        img_map = lambda b: (b, 0, 0, 0)

    body = functools.partial(
        _scale_kernel, s=s, h=h, w=w, f=f, R=R, K=K, bw=bw,
        hranges0=hranges0, chunks=chunks)

    out = pl.pallas_call(
        body,
        grid=(B,),
        in_specs=[
            pl.BlockSpec(img_block, img_map),
            pl.BlockSpec(img_block, img_map),
            pl.BlockSpec((1, 2, h, w), lambda b: (b, 0, 0, 0)),
            pl.BlockSpec((h, 16), lambda b: (0, 0)),
            pl.BlockSpec((8, w), lambda b: (0, 0)),
        ],
        out_specs=pl.BlockSpec((1, 1, _LANES), lambda b: (b, 0, 0)),
        out_shape=jax.ShapeDtypeStruct((B, 1, _LANES), jnp.float32),
        compiler_params=pltpu.CompilerParams(
            dimension_semantics=("parallel",)),
    )(il_in, ir_in, disp, jnp.asarray(rc), jnp.asarray(xw))
    return jnp.sum(out[:, 0, :8], axis=0)


def kernel(images_left, images_right, disp0, disp1, disp2, disp3):
    B, _, H, W = images_left.shape
    disps = [disp0, disp1, disp2, disp3]
    apps, smooths, lrcs = [], [], []
    for s in range(4):
        h, w = H >> s, W >> s
        S = _scale_call(s, images_left, images_right, disps[s])
        n_ssim = np.float32(B * 3 * (h - 2) * (w - 2))
        n_x = np.float32(B * 3 * h * (w - 1))
        n_y = np.float32(B * 3 * (h - 1) * w)
        n_d = np.float32(B * h * w)
        apps.append((S[0] + S[1]) / n_ssim)
        smooths.append(((S[2] + S[4]) / n_x + (S[3] + S[5]) / n_y)
                       / np.float32((s + 1) ** 2))
        lrcs.append((S[6] + S[7]) / n_d)
    return jnp.stack([
        jnp.mean(jnp.stack(apps)),
        jnp.mean(jnp.stack(smooths)),
        jnp.mean(jnp.stack(lrcs)),
    ])
